# baseline (device time: 79341 ns/iter reference)
import jax
import jax.numpy as jnp
from jax import lax
from jax.experimental import pallas as pl
from jax.experimental.pallas import tpu as pltpu

M = 4096
BLK = 2048
HALF = 1024
D = 2048

SIZES = [128] * 8
OFFS = [sum(SIZES[:i]) for i in range(len(SIZES))]
C = len(SIZES)
CHMAX = max(SIZES)
assert sum(SIZES) == HALF


def kernel(partial, gamma):
    g = gamma.reshape(1, D)

    def body(p_ref, g_ref, dummy_ref, out_ref, my_f32, stage, xsend, xrecv, ysend, yrecv,
             ostage, my_sem, stage_sems, xsend_sems, xrecv_sems, ysend_sems,
             yrecv_sems, out_sems):
        my_x = lax.axis_index("x")
        my_y = lax.axis_index("y")
        peer_x = 1 - my_x
        peer_y = 1 - my_y

        my_rows = my_x * BLK + my_y * HALF
        send_rows = peer_x * BLK + my_y * HALF

        stage_cps = [
            pltpu.make_async_copy(
                p_ref.at[0, pl.ds(send_rows + OFFS[s], SIZES[s]), :],
                stage.at[s, pl.ds(0, SIZES[s]), :], stage_sems.at[s],
            )
            for s in range(2)
        ]
        for cp in stage_cps:
            cp.start()
        my_cp = pltpu.make_async_copy(
            p_ref.at[0, pl.ds(my_rows, HALF), :], my_f32, my_sem
        )
        my_cp.start()

        barrier = pltpu.get_barrier_semaphore()
        pl.semaphore_signal(
            barrier, inc=1,
            device_id=(peer_x, my_y), device_id_type=pl.DeviceIdType.MESH,
        )
        pl.semaphore_signal(
            barrier, inc=1,
            device_id=(my_x, peer_y), device_id_type=pl.DeviceIdType.MESH,
        )
        pl.semaphore_wait(barrier, 2)

        x_rdmas = []
        for c in range(C):
            stage_cps[c].wait()
            sl = pl.ds(OFFS[c], SIZES[c])
            xsend[sl, :] = stage[c % 2, pl.ds(0, SIZES[c]), :].astype(jnp.bfloat16)
            if c + 2 < C:
                cpn = pltpu.make_async_copy(
                    p_ref.at[0, pl.ds(send_rows + OFFS[c + 2], SIZES[c + 2]), :],
                    stage.at[c % 2, pl.ds(0, SIZES[c + 2]), :],
                    stage_sems.at[c % 2],
                )
                cpn.start()
                stage_cps.append(cpn)
            rd = pltpu.make_async_remote_copy(
                src_ref=xsend.at[sl, :],
                dst_ref=xrecv.at[sl, :],
                send_sem=xsend_sems.at[c],
                recv_sem=xrecv_sems.at[c],
                device_id=(peer_x, my_y),
                device_id_type=pl.DeviceIdType.MESH,
            )
            rd.start()
            x_rdmas.append(rd)

        my_cp.wait()
        y_rdmas = []
        out_cps = {}
        for c in range(C + 1):
            if c < C:
                x_rdmas[c].wait_recv()
                sl = pl.ds(OFFS[c], SIZES[c])
                yv = my_f32[sl, :] + xrecv[sl, :].astype(jnp.float32)
                ss = jnp.sum(yv * yv, axis=-1, keepdims=True)
                r = lax.rsqrt(ss / D + 1e-6)
                o = yv * r * g_ref[...]
                ostage[sl, :] = o
                ysend[sl, :] = o.astype(jnp.bfloat16)
                yr = pltpu.make_async_remote_copy(
                    src_ref=ysend.at[sl, :],
                    dst_ref=yrecv.at[sl, :],
                    send_sem=ysend_sems.at[c],
                    recv_sem=yrecv_sems.at[c],
                    device_id=(my_x, peer_y),
                    device_id_type=pl.DeviceIdType.MESH,
                )
                yr.start()
                y_rdmas.append(yr)
                ocp = pltpu.make_async_copy(
                    ostage.at[sl, :],
                    out_ref.at[pl.ds(my_y * HALF + OFFS[c], SIZES[c]), :],
                    out_sems.at[c],
                )
                ocp.start()
                out_cps[c] = ocp
            d = c - 1
            if d >= 0:
                y_rdmas[d].wait_recv()
                sld = pl.ds(HALF + OFFS[d], SIZES[d])
                ostage[sld, :] = yrecv[pl.ds(OFFS[d], SIZES[d]), :].astype(
                    jnp.float32
                )
                ocp = pltpu.make_async_copy(
                    ostage.at[sld, :],
                    out_ref.at[pl.ds(peer_y * HALF + OFFS[d], SIZES[d]), :],
                    out_sems.at[C + d],
                )
                ocp.start()
                out_cps[C + d] = ocp

        for c in range(2 * C):
            out_cps[c].wait()
        for c in range(C):
            x_rdmas[c].wait_send()
            y_rdmas[c].wait_send()

    return pl.pallas_call(
        body,
        out_shape=jax.ShapeDtypeStruct((BLK, D), jnp.float32),
        in_specs=[
            pl.BlockSpec(memory_space=pl.ANY),
            pl.BlockSpec(memory_space=pltpu.VMEM),
            pl.BlockSpec(memory_space=pl.ANY),
        ],
        out_specs=pl.BlockSpec(memory_space=pl.ANY),
        input_output_aliases={2: 0},
        scratch_shapes=[
            pltpu.VMEM((HALF, D), jnp.float32),
            pltpu.VMEM((2, CHMAX, D), jnp.float32),
            pltpu.VMEM((HALF, D), jnp.bfloat16),
            pltpu.VMEM((HALF, D), jnp.bfloat16),
            pltpu.VMEM((HALF, D), jnp.bfloat16),
            pltpu.VMEM((HALF, D), jnp.bfloat16),
            pltpu.VMEM((BLK, D), jnp.float32),
            pltpu.SemaphoreType.DMA,
            pltpu.SemaphoreType.DMA((2,)),
            pltpu.SemaphoreType.DMA((C,)),
            pltpu.SemaphoreType.DMA((C,)),
            pltpu.SemaphoreType.DMA((C,)),
            pltpu.SemaphoreType.DMA((C,)),
            pltpu.SemaphoreType.DMA((2 * C,)),
        ],
        compiler_params=pltpu.CompilerParams(
            collective_id=0,
            vmem_limit_bytes=128 * 1024 * 1024,
        ),
    )(partial, g, jnp.zeros((BLK, D), jnp.float32))


# device time: 72955 ns/iter; 1.0875x vs baseline; 1.0875x over previous
import jax
import jax.numpy as jnp
from jax import lax
from jax.experimental import pallas as pl
from jax.experimental.pallas import tpu as pltpu

M = 4096
BLK = 2048
HALF = 1024
D = 2048

SIZES = [128] * 8
OFFS = [sum(SIZES[:i]) for i in range(len(SIZES))]
C = len(SIZES)
CHMAX = max(SIZES)
assert sum(SIZES) == HALF


def kernel(partial, gamma):
    g = gamma.reshape(1, D)

    def body(p_ref, g_ref, out_ref, my_f32, stage, xsend, xrecv, ysend, yrecv,
             ostage, my_sem, stage_sems, xsend_sems, xrecv_sems, ysend_sems,
             yrecv_sems, out_sems):
        my_x = lax.axis_index("x")
        my_y = lax.axis_index("y")
        peer_x = 1 - my_x
        peer_y = 1 - my_y

        my_rows = my_x * BLK + my_y * HALF
        send_rows = peer_x * BLK + my_y * HALF

        stage_cps = [
            pltpu.make_async_copy(
                p_ref.at[0, pl.ds(send_rows + OFFS[s], SIZES[s]), :],
                stage.at[s, pl.ds(0, SIZES[s]), :], stage_sems.at[s],
            )
            for s in range(2)
        ]
        for cp in stage_cps:
            cp.start()
        my_cp = pltpu.make_async_copy(
            p_ref.at[0, pl.ds(my_rows, HALF), :], my_f32, my_sem
        )
        my_cp.start()

        barrier = pltpu.get_barrier_semaphore()
        pl.semaphore_signal(
            barrier, inc=1,
            device_id=(peer_x, my_y), device_id_type=pl.DeviceIdType.MESH,
        )
        pl.semaphore_signal(
            barrier, inc=1,
            device_id=(my_x, peer_y), device_id_type=pl.DeviceIdType.MESH,
        )
        pl.semaphore_wait(barrier, 2)

        x_rdmas = []
        for c in range(C):
            stage_cps[c].wait()
            sl = pl.ds(OFFS[c], SIZES[c])
            xsend[sl, :] = stage[c % 2, pl.ds(0, SIZES[c]), :].astype(jnp.bfloat16)
            if c + 2 < C:
                cpn = pltpu.make_async_copy(
                    p_ref.at[0, pl.ds(send_rows + OFFS[c + 2], SIZES[c + 2]), :],
                    stage.at[c % 2, pl.ds(0, SIZES[c + 2]), :],
                    stage_sems.at[c % 2],
                )
                cpn.start()
                stage_cps.append(cpn)
            rd = pltpu.make_async_remote_copy(
                src_ref=xsend.at[sl, :],
                dst_ref=xrecv.at[sl, :],
                send_sem=xsend_sems.at[c],
                recv_sem=xrecv_sems.at[c],
                device_id=(peer_x, my_y),
                device_id_type=pl.DeviceIdType.MESH,
            )
            rd.start()
            x_rdmas.append(rd)

        my_cp.wait()
        y_rdmas = []
        out_cps = {}
        for c in range(C + 1):
            if c < C:
                x_rdmas[c].wait_recv()
                sl = pl.ds(OFFS[c], SIZES[c])
                yv = my_f32[sl, :] + xrecv[sl, :].astype(jnp.float32)
                ss = jnp.sum(yv * yv, axis=-1, keepdims=True)
                r = lax.rsqrt(ss / D + 1e-6)
                o = yv * r * g_ref[...]
                ostage[sl, :] = o
                ysend[sl, :] = o.astype(jnp.bfloat16)
                yr = pltpu.make_async_remote_copy(
                    src_ref=ysend.at[sl, :],
                    dst_ref=yrecv.at[sl, :],
                    send_sem=ysend_sems.at[c],
                    recv_sem=yrecv_sems.at[c],
                    device_id=(my_x, peer_y),
                    device_id_type=pl.DeviceIdType.MESH,
                )
                yr.start()
                y_rdmas.append(yr)
                ocp = pltpu.make_async_copy(
                    ostage.at[sl, :],
                    out_ref.at[pl.ds(my_y * HALF + OFFS[c], SIZES[c]), :],
                    out_sems.at[c],
                )
                ocp.start()
                out_cps[c] = ocp
            d = c - 1
            if d >= 0:
                y_rdmas[d].wait_recv()
                sld = pl.ds(HALF + OFFS[d], SIZES[d])
                ostage[sld, :] = yrecv[pl.ds(OFFS[d], SIZES[d]), :].astype(
                    jnp.float32
                )
                ocp = pltpu.make_async_copy(
                    ostage.at[sld, :],
                    out_ref.at[pl.ds(peer_y * HALF + OFFS[d], SIZES[d]), :],
                    out_sems.at[C + d],
                )
                ocp.start()
                out_cps[C + d] = ocp

        for c in range(2 * C):
            out_cps[c].wait()
        for c in range(C):
            x_rdmas[c].wait_send()
            y_rdmas[c].wait_send()

    return pl.pallas_call(
        body,
        out_shape=jax.ShapeDtypeStruct((BLK, D), jnp.float32),
        in_specs=[
            pl.BlockSpec(memory_space=pl.ANY),
            pl.BlockSpec(memory_space=pltpu.VMEM),
        ],
        out_specs=pl.BlockSpec(memory_space=pl.ANY),
        scratch_shapes=[
            pltpu.VMEM((HALF, D), jnp.float32),
            pltpu.VMEM((2, CHMAX, D), jnp.float32),
            pltpu.VMEM((HALF, D), jnp.bfloat16),
            pltpu.VMEM((HALF, D), jnp.bfloat16),
            pltpu.VMEM((HALF, D), jnp.bfloat16),
            pltpu.VMEM((HALF, D), jnp.bfloat16),
            pltpu.VMEM((BLK, D), jnp.float32),
            pltpu.SemaphoreType.DMA,
            pltpu.SemaphoreType.DMA((2,)),
            pltpu.SemaphoreType.DMA((C,)),
            pltpu.SemaphoreType.DMA((C,)),
            pltpu.SemaphoreType.DMA((C,)),
            pltpu.SemaphoreType.DMA((C,)),
            pltpu.SemaphoreType.DMA((2 * C,)),
        ],
        compiler_params=pltpu.CompilerParams(
            collective_id=0,
            vmem_limit_bytes=128 * 1024 * 1024,
        ),
    )(partial, g)
